# Initial kernel scaffold; baseline (speedup 1.0000x reference)
import jax
import jax.numpy as jnp
from jax.experimental import pallas as pl


def kernel(inputs, embed):
    raise NotImplementedError("write your pallas kernel here")


# speedup vs baseline: 1.8510x; 1.8510x over previous
import math

import jax
import jax.numpy as jnp
from jax import lax
from jax.experimental import pallas as pl
from jax.experimental.pallas import tpu as pltpu

NB = 16
C = 64
HW = 1024
NE = 1024
N_TOK = NB * HW


def _vq_body(x_ref, e_ref, zq_ref, loss_ref, perp_ref, hist_ref, acc_ref):
    b = pl.program_id(0)

    @pl.when(b == 0)
    def _init():
        acc_ref[0] = 0.0
        hist_ref[...] = jnp.zeros_like(hist_ref)

    x = x_ref[0]
    emb = e_ref[...]

    mm = lax.dot_general(emb, x, (((1,), (0,)), ((), ())),
                         preferred_element_type=jnp.float32)
    e2 = jnp.sum(emb * emb, axis=1, keepdims=True)
    xn = jnp.sum(x * x, axis=0, keepdims=True)
    d2 = (xn + e2) - 2.0 * mm

    mind = jnp.min(d2, axis=0, keepdims=True)
    iota = lax.broadcasted_iota(jnp.int32, (NE, HW), 0)
    idx = jnp.min(jnp.where(d2 == mind, iota, NE), axis=0, keepdims=True)
    onehot = (iota == idx).astype(jnp.float32)

    zq = lax.dot_general(emb, onehot, (((0,), (0,)), ((), ())),
                         preferred_element_type=jnp.float32)
    diff = zq - x
    zq_ref[0] = x + diff

    acc_ref[0] += jnp.sum(diff * diff)
    hist_ref[...] += jnp.sum(onehot, axis=1, keepdims=True)

    @pl.when(b == NB - 1)
    def _fini():
        loss_ref[0, 0] = 1.25 * acc_ref[0] / (NB * C * HW)
        probs = hist_ref[...] * (1.0 / N_TOK)
        ent = -jnp.sum(probs * jnp.log(probs + 1e-10))
        perp_ref[0, 0] = jnp.exp(ent)


def _vq_call(x3, embed, interpret=False):
    return pl.pallas_call(
        _vq_body,
        grid=(NB,),
        in_specs=[
            pl.BlockSpec((1, C, HW), lambda b: (b, 0, 0)),
            pl.BlockSpec((NE, C), lambda b: (0, 0)),
        ],
        out_specs=[
            pl.BlockSpec((1, C, HW), lambda b: (b, 0, 0)),
            pl.BlockSpec(memory_space=pltpu.SMEM),
            pl.BlockSpec(memory_space=pltpu.SMEM),
        ],
        out_shape=[
            jax.ShapeDtypeStruct((NB, C, HW), jnp.float32),
            jax.ShapeDtypeStruct((1, 1), jnp.float32),
            jax.ShapeDtypeStruct((1, 1), jnp.float32),
        ],
        scratch_shapes=[
            pltpu.VMEM((NE, 1), jnp.float32),
            pltpu.SMEM((1,), jnp.float32),
        ],
        interpret=interpret,
    )(x3, embed)


def kernel(inputs, embed):
    x3 = inputs.reshape(NB, C, HW)
    zq, loss, perp = _vq_call(x3, embed)
    z_q_out = zq.reshape(NB, C, 32, 32)
    kldiv_r = math.log(NE) * HW * jnp.ones((NB, 1), dtype=jnp.float32)
    return (z_q_out, loss[0, 0], kldiv_r, perp[0, 0])
